# baseline (speedup 1.0000x reference)
import jax
import jax.numpy as jnp
from jax.experimental import pallas as pl
from jax.experimental.pallas import tpu as pltpu

_BK = 2048


def _pool_kernel(p_ref, x_ref, o_ref, acc_ref):
    c = pl.program_id(0)
    nc = pl.num_programs(0)
    nb = acc_ref.shape[0]
    s = nb // 2

    @pl.when(c == 0)
    def _():
        acc_ref[...] = jnp.zeros_like(acc_ref)

    p = p_ref[...]
    row_ids = jax.lax.broadcasted_iota(jnp.int32, (nb, _BK), 1) + c * _BK
    mask = (row_ids < p).astype(jnp.bfloat16)
    x = x_ref[...]
    x_hi = x.astype(jnp.bfloat16)
    x_lo = (x - x_hi.astype(jnp.float32)).astype(jnp.bfloat16)
    dn = (((1,), (0,)), ((), ()))
    acc_ref[...] += (
        jax.lax.dot_general(mask, x_hi, dn, preferred_element_type=jnp.float32)
        + jax.lax.dot_general(mask, x_lo, dn, preferred_element_type=jnp.float32))

    @pl.when(c == nc - 1)
    def _():
        acc = acc_ref[...]
        cnt = (p[s:] - p[:s]).astype(jnp.float32)
        o_ref[...] = (acc[s:, :] - acc[:s, :]) / cnt + jnp.float32(0.006)


def kernel(input, graph):
    n, d = input.shape
    s = graph.shape[0]
    g = graph.astype(jnp.int32)
    p = jnp.concatenate([g[:, 0], g[:, 1] + 1]).reshape(2 * s, 1)
    return pl.pallas_call(
        _pool_kernel,
        grid=(n // _BK,),
        in_specs=[
            pl.BlockSpec((2 * s, 1), lambda c: (0, 0)),
            pl.BlockSpec((_BK, d), lambda c: (c, 0)),
        ],
        out_specs=pl.BlockSpec((s, d), lambda c: (0, 0)),
        out_shape=jax.ShapeDtypeStruct((s, d), jnp.float32),
        scratch_shapes=[pltpu.VMEM((2 * s, d), jnp.float32)],
    )(p, input)


# speedup vs baseline: 1.4255x; 1.4255x over previous
import jax
import jax.numpy as jnp
from jax.experimental import pallas as pl
from jax.experimental.pallas import tpu as pltpu

_BK = 4096


def _pool_kernel(p_ref, x_ref, o_ref, acc_ref):
    c = pl.program_id(0)
    nc = pl.num_programs(0)
    nb = acc_ref.shape[0]
    s = nb // 2

    @pl.when(c == 0)
    def _():
        acc_ref[...] = jnp.zeros_like(acc_ref)

    p = p_ref[...]
    row_ids = jax.lax.broadcasted_iota(jnp.int32, (nb, _BK), 1) + c * _BK
    mask = (row_ids < p).astype(jnp.float32)
    acc_ref[...] += jax.lax.dot_general(
        mask, x_ref[...], (((1,), (0,)), ((), ())),
        preferred_element_type=jnp.float32)

    @pl.when(c == nc - 1)
    def _():
        acc = acc_ref[...]
        cnt = (p[s:] - p[:s]).astype(jnp.float32)
        o_ref[...] = (acc[s:, :] - acc[:s, :]) / cnt + jnp.float32(0.006)


def kernel(input, graph):
    n, d = input.shape
    s = graph.shape[0]
    g = graph.astype(jnp.int32)
    p = jnp.concatenate([g[:, 0], g[:, 1] + 1]).reshape(2 * s, 1)
    return pl.pallas_call(
        _pool_kernel,
        grid=(n // _BK,),
        in_specs=[
            pl.BlockSpec((2 * s, 1), lambda c: (0, 0)),
            pl.BlockSpec((_BK, d), lambda c: (c, 0)),
        ],
        out_specs=pl.BlockSpec((s, d), lambda c: (0, 0)),
        out_shape=jax.ShapeDtypeStruct((s, d), jnp.float32),
        scratch_shapes=[pltpu.VMEM((2 * s, d), jnp.float32)],
    )(p, input)


# speedup vs baseline: 1.5672x; 1.0994x over previous
import jax
import jax.numpy as jnp
from jax.experimental import pallas as pl
from jax.experimental.pallas import tpu as pltpu

_BK = 8192


def _pool_kernel(p_ref, x_ref, o_ref, acc_ref):
    c = pl.program_id(0)
    nc = pl.num_programs(0)
    nb = acc_ref.shape[0]
    s = nb // 2

    @pl.when(c == 0)
    def _():
        acc_ref[...] = jnp.zeros_like(acc_ref)

    p = p_ref[...]
    row_ids = jax.lax.broadcasted_iota(jnp.int32, (nb, _BK), 1) + c * _BK
    mask = (row_ids < p).astype(jnp.float32)
    acc_ref[...] += jax.lax.dot_general(
        mask, x_ref[...], (((1,), (0,)), ((), ())),
        preferred_element_type=jnp.float32)

    @pl.when(c == nc - 1)
    def _():
        acc = acc_ref[...]
        cnt = (p[s:] - p[:s]).astype(jnp.float32)
        o_ref[...] = (acc[s:, :] - acc[:s, :]) / cnt + jnp.float32(0.006)


def kernel(input, graph):
    n, d = input.shape
    s = graph.shape[0]
    g = graph.astype(jnp.int32)
    p = jnp.concatenate([g[:, 0], g[:, 1] + 1]).reshape(2 * s, 1)
    return pl.pallas_call(
        _pool_kernel,
        grid=(n // _BK,),
        in_specs=[
            pl.BlockSpec((2 * s, 1), lambda c: (0, 0)),
            pl.BlockSpec((_BK, d), lambda c: (c, 0)),
        ],
        out_specs=pl.BlockSpec((s, d), lambda c: (0, 0)),
        out_shape=jax.ShapeDtypeStruct((s, d), jnp.float32),
        scratch_shapes=[pltpu.VMEM((2 * s, d), jnp.float32)],
    )(p, input)
